# baseline (speedup 1.0000x reference)
import functools

import jax
import jax.numpy as jnp
from jax import lax
from jax.experimental import pallas as pl
from jax.experimental.pallas import tpu as pltpu
from jax.experimental.pallas import tpu_sc as plsc

N_FACTORS = 32
BATCH = 16384
V_ROWS = 1000000
TCOL = 128
NTCOLS = (V_ROWS + TCOL - 1) // TCOL
NC = 2
NS = 16
NW = NC * NS
RNG = (NTCOLS + NW - 1) // NW
WCOL = 4
WELEM = WCOL * TCOL
NWIN = (RNG + WCOL - 1) // WCOL
PITCH = WELEM + 1
LANES = 16
CAP = BATCH + 2 * LANES
STAGE_ROWS = BATCH + NW
B_PER_W = BATCH // NW


def _sweep(idx_hbm, tab_hbm, dst_hbm, aux, uo, bo, wlb, slab, stage, sidx,
           sem_f, sem_f2, sem_s, sem_s2, wid, w0):
    lane = jnp.arange(LANES, dtype=jnp.int32)
    f_lo = lane
    f_hi = lane + LANES

    pltpu.sync_copy(idx_hbm, aux.at[pl.ds(0, BATCH)])

    def scan(v, cnt):
        u_vec = aux[pl.ds(v * LANES, LANES)]
        tc = lax.shift_right_logical(u_vec, 7)
        m = (tc >= w0) & (tc < w0 + RNG)
        plsc.store_compressed(uo.at[pl.ds(cnt, LANES)], u_vec, mask=m)
        b_vec = jnp.broadcast_to(v * LANES, (LANES,)) + lane
        plsc.store_compressed(bo.at[pl.ds(cnt, LANES)], b_vec, mask=m)
        return cnt + plsc.all_reduce_population_count(m)[0]

    cnt = lax.fori_loop(0, BATCH // LANES, scan, jnp.int32(0))
    uo[pl.ds(cnt, LANES)] = jnp.broadcast_to(jnp.int32(0x3FFFFFFF), (LANES,))
    nv = lax.shift_right_logical(cnt + LANES - 1, 4)

    dummy_row = jnp.int32(BATCH) + wid
    sems_f = (sem_f, sem_f2)
    sems_s = (sem_s, sem_s2)

    def cwc_of(w):
        return jnp.minimum(w0 + w * WCOL, NTCOLS - WCOL)

    def fire(w, par):
        off = pl.multiple_of(cwc_of(w) * TCOL, TCOL)
        pltpu.async_copy(tab_hbm.at[:, pl.ds(off, WELEM)],
                         slab.at[par, :, pl.ds(0, WELEM)], sems_f[par])

    def drain_fetch(par):
        pltpu.make_async_copy(
            tab_hbm.at[:, pl.ds(0, WELEM)],
            slab.at[par, :, pl.ds(0, WELEM)], sems_f[par]).wait()

    def drain_scatter(par):
        pltpu.make_async_copy(
            dst_hbm.at[pl.ds(0, LANES)], stage.at[par], sems_s[par]).wait()

    fire(jnp.int32(0), 0)
    fire(jnp.int32(1), 1)

    def win_pair(h, _):
        for par in (0, 1):
            w = h * 2 + par
            drain_fetch(par)
            cwc = cwc_of(w)
            base_u = cwc * TCOL

            def rescan(v, wcnt, cwc=cwc, base_u=base_u):
                u_vec = uo[pl.ds(v * LANES, LANES)]
                tc = lax.shift_right_logical(u_vec, 7)
                wm = (tc >= cwc) & (tc < cwc + WCOL)
                off_vec = u_vec - jnp.broadcast_to(base_u, (LANES,))
                plsc.store_compressed(aux.at[pl.ds(wcnt, LANES)], off_vec,
                                      mask=wm)
                b_vec = bo[pl.ds(v * LANES, LANES)]
                plsc.store_compressed(wlb.at[pl.ds(wcnt, LANES)], b_vec,
                                      mask=wm)
                return wcnt + plsc.all_reduce_population_count(wm)[0]

            wcnt = lax.fori_loop(0, nv, rescan, jnp.int32(0))
            nblk = lax.shift_right_logical(wcnt + LANES - 1, 4)

            def blk_pair(hb, _, par=par, wcnt=wcnt, nblk=nblk):
                for sp in (0, 1):
                    blk = hb * 2 + sp

                    @pl.when(blk < nblk)
                    def _do(blk=blk, sp=sp, par=par, wcnt=wcnt):
                        @pl.when(blk >= 2)
                        def _dr():
                            drain_scatter(sp)

                        offv = aux[pl.ds(blk * LANES, LANES)] & (WELEM - 1)
                        bv = wlb[pl.ds(blk * LANES, LANES)]
                        pos = jnp.broadcast_to(blk * LANES, (LANES,)) + lane
                        sidx[sp, pl.ds(0, LANES)] = jnp.where(
                            pos < wcnt, bv, dummy_row)
                        for l in range(LANES):
                            go = jnp.broadcast_to(offv[l], (LANES,))
                            c0 = plsc.load_gather(slab.at[par], [f_lo, go])
                            c1 = plsc.load_gather(slab.at[par], [f_hi, go])
                            stage[sp, l, pl.ds(0, LANES)] = c0
                            stage[sp, l, pl.ds(LANES, LANES)] = c1
                        pltpu.async_copy(stage.at[sp],
                                         dst_hbm.at[sidx.at[sp]], sems_s[sp])
                return 0

            lax.fori_loop(0, (nblk + 1) >> 1, blk_pair, 0)

            @pl.when(nblk >= 1)
            def _fd0():
                drain_scatter(0)

            @pl.when(nblk >= 2)
            def _fd1():
                drain_scatter(1)

            @pl.when(w + 2 < NWIN)
            def _refill(w=w, par=par):
                fire(w + 2, par)
        return 0

    lax.fori_loop(0, NWIN // 2, win_pair, 0)


def _k1_body(user_hbm, item_hbm, uT_hbm, iT_hbm, pr_hbm, qr_hbm,
             aux, uo, bo, wlb, slab, stage, sidx,
             sem_f, sem_f2, sem_s, sem_s2):
    wid = lax.axis_index("s") * NC + lax.axis_index("c")
    w0 = wid * RNG
    _sweep(user_hbm, uT_hbm, pr_hbm, aux, uo, bo, wlb, slab, stage, sidx,
           sem_f, sem_f2, sem_s, sem_s2, wid, w0)
    _sweep(item_hbm, iT_hbm, qr_hbm, aux, uo, bo, wlb, slab, stage, sidx,
           sem_f, sem_f2, sem_s, sem_s2, wid, w0)


def _k2_body(pr_hbm, qr_hbm, hw_hbm, hb_hbm, out_hbm,
             ch_u, ch_i, w_v, b_v, out_v, sem0, sem1):
    wid = lax.axis_index("s") * NC + lax.axis_index("c")
    base = wid * B_PER_W
    sems = (sem0, sem1)
    pltpu.sync_copy(hw_hbm.at[0], w_v)
    pltpu.sync_copy(hb_hbm, b_v.at[pl.ds(0, 1)])
    w0 = w_v[pl.ds(0, LANES)]
    w1 = w_v[pl.ds(LANES, LANES)]
    b = b_v[pl.ds(0, LANES)][0]
    lane = jnp.arange(LANES, dtype=jnp.int32)
    NCH = B_PER_W // 128

    def fire(c):
        p = c & 1
        return (
            pltpu.async_copy(pr_hbm.at[pl.ds(base + c * 128, 128)],
                             ch_u.at[p], sems[p]),
            pltpu.async_copy(qr_hbm.at[pl.ds(base + c * 128, 128)],
                             ch_i.at[p], sems[p]),
        )

    inflight = {0: fire(0)}
    for c in range(NCH):
        if c + 1 < NCH:
            inflight[c + 1] = fire(c + 1)
        for cp in inflight.pop(c):
            cp.wait()
        p = c & 1

        def group(g, _, c=c, p=p):
            acc = jnp.zeros((LANES,), jnp.float32)
            for j in range(LANES):
                r = g * LANES + j
                p0 = ch_u[p, r, pl.ds(0, LANES)]
                p1 = ch_u[p, r, pl.ds(LANES, LANES)]
                q0 = ch_i[p, r, pl.ds(0, LANES)]
                q1 = ch_i[p, r, pl.ds(LANES, LANES)]
                s = p0 * q0 * w0 + p1 * q1 * w1
                acc = jnp.where(lane == j, jnp.sum(s), acc)
            out_v[pl.ds(c * 128 + g * LANES, LANES)] = acc + b
            return 0

        lax.fori_loop(0, 128 // LANES, group, 0)

    pltpu.sync_copy(out_v, out_hbm.at[pl.ds(base, B_PER_W)])


@jax.jit
def _gmf(user, item, user_emb, item_emb, h_w, h_b):
    mesh = plsc.VectorSubcoreMesh(core_axis_name="c", subcore_axis_name="s")
    k1 = functools.partial(
        pl.kernel,
        mesh=mesh,
        out_type=(
            jax.ShapeDtypeStruct((STAGE_ROWS, TCOL), jnp.float32),
            jax.ShapeDtypeStruct((STAGE_ROWS, TCOL), jnp.float32),
        ),
        scratch_types=[
            pltpu.VMEM((CAP,), jnp.int32),
            pltpu.VMEM((CAP,), jnp.int32),
            pltpu.VMEM((CAP,), jnp.int32),
            pltpu.VMEM((CAP,), jnp.int32),
            pltpu.VMEM((2, N_FACTORS, PITCH), jnp.float32),
            pltpu.VMEM((2, LANES, TCOL), jnp.float32),
            pltpu.VMEM((2, LANES), jnp.int32),
            pltpu.SemaphoreType.DMA,
            pltpu.SemaphoreType.DMA,
            pltpu.SemaphoreType.DMA,
            pltpu.SemaphoreType.DMA,
        ],
        compiler_params=pltpu.CompilerParams(needs_layout_passes=False),
    )(_k1_body)
    pr, qr = k1(user, item, user_emb.T, item_emb.T)

    k2 = functools.partial(
        pl.kernel,
        mesh=mesh,
        out_type=jax.ShapeDtypeStruct((BATCH,), jnp.float32),
        scratch_types=[
            pltpu.VMEM((2, 128, TCOL), jnp.float32),
            pltpu.VMEM((2, 128, TCOL), jnp.float32),
            pltpu.VMEM((N_FACTORS,), jnp.float32),
            pltpu.VMEM((LANES,), jnp.float32),
            pltpu.VMEM((B_PER_W,), jnp.float32),
            pltpu.SemaphoreType.DMA,
            pltpu.SemaphoreType.DMA,
        ],
        compiler_params=pltpu.CompilerParams(needs_layout_passes=False),
    )(_k2_body)
    return k2(pr, qr, h_w, h_b)


def kernel(user, item, user_emb, item_emb, h_w, h_b):
    return _gmf(user, item, user_emb, item_emb, h_w, h_b)


# speedup vs baseline: 1.3061x; 1.3061x over previous
import functools

import jax
import jax.numpy as jnp
from jax import lax
from jax.experimental import pallas as pl
from jax.experimental.pallas import tpu as pltpu
from jax.experimental.pallas import tpu_sc as plsc

N_FACTORS = 32
BATCH = 16384
V_ROWS = 1000000
TCOL = 128
NTCOLS = (V_ROWS + TCOL - 1) // TCOL
NC = 2
NS = 16
NW = NC * NS
RNG = (NTCOLS + NW - 1) // NW
WCOL = 8
WELEM = WCOL * TCOL
NWIN = 32
PITCH = WELEM + 1
LANES = 16
CAP = BATCH + 2 * LANES
STAGE_ROWS = BATCH + NW
B_PER_W = BATCH // NW


def _sweep(idx_hbm, tab_hbm, dst_hbm, aux, uo, slab, stage, sidx,
           sem_f, sem_f2, sem_s, sem_s2, wid, w0):
    lane = jnp.arange(LANES, dtype=jnp.int32)
    f_lo = lane
    f_hi = lane + LANES

    pltpu.sync_copy(idx_hbm, aux.at[pl.ds(0, BATCH)])

    def scan(v, cnt):
        u_vec = aux[pl.ds(v * LANES, LANES)]
        tc = lax.shift_right_logical(u_vec, 7)
        m = (tc >= w0) & (tc < w0 + RNG)
        b_vec = jnp.broadcast_to(v * LANES, (LANES,)) + lane
        e_vec = (lax.shift_left(tc - w0, 21)
                 | lax.shift_left(u_vec & (TCOL - 1), 14) | b_vec)
        plsc.store_compressed(uo.at[pl.ds(cnt, LANES)], e_vec, mask=m)
        return cnt + plsc.all_reduce_population_count(m)[0]

    cnt = lax.fori_loop(0, BATCH // LANES, scan, jnp.int32(0))
    uo[pl.ds(cnt, LANES)] = jnp.broadcast_to(jnp.int32(0x7F000000), (LANES,))
    nv = lax.shift_right_logical(cnt + LANES - 1, 4)

    dummy_row = jnp.int32(BATCH) + wid
    sems_f = (sem_f, sem_f2)
    sems_s = (sem_s, sem_s2)

    def cwc_of(w):
        return jnp.minimum(w0 + w * WCOL, NTCOLS - WCOL)

    def fire(w, par):
        off = pl.multiple_of(cwc_of(w) * TCOL, TCOL)
        pltpu.async_copy(tab_hbm.at[:, pl.ds(off, WELEM)],
                         slab.at[par, :, pl.ds(0, WELEM)], sems_f[par])

    def drain_fetch(par):
        pltpu.make_async_copy(
            tab_hbm.at[:, pl.ds(0, WELEM)],
            slab.at[par, :, pl.ds(0, WELEM)], sems_f[par]).wait()

    def drain_scatter(par):
        pltpu.make_async_copy(
            dst_hbm.at[pl.ds(0, LANES)], stage.at[par], sems_s[par]).wait()

    fire(jnp.int32(0), 0)
    fire(jnp.int32(1), 1)

    def win_pair(h, prev):
        for par in (0, 1):
            w = h * 2 + par
            drain_fetch(par)
            cwc = cwc_of(w)
            lo = cwc - w0

            def rescan(v, wcnt, lo=lo):
                e_vec = uo[pl.ds(v * LANES, LANES)]
                tcr = lax.shift_right_logical(e_vec, 21)
                wm = (tcr >= lo) & (tcr < lo + WCOL)
                off = (lax.shift_left(tcr - lo, 7)
                       | (lax.shift_right_logical(e_vec, 14) & (TCOL - 1)))
                wl = lax.shift_left(off, 14) | (e_vec & 16383)
                plsc.store_compressed(aux.at[pl.ds(wcnt, LANES)], wl, mask=wm)
                return wcnt + plsc.all_reduce_population_count(wm)[0]

            wcnt = lax.fori_loop(0, nv, rescan, jnp.int32(0))
            nblk = lax.shift_right_logical(wcnt + LANES - 1, 4)

            @pl.when(prev >= 1)
            def _pd0():
                drain_scatter(0)

            @pl.when(prev >= 2)
            def _pd1():
                drain_scatter(1)

            def blk_pair(hb, _, par=par, wcnt=wcnt, nblk=nblk):
                for sp in (0, 1):
                    blk = hb * 2 + sp

                    @pl.when(blk < nblk)
                    def _do(blk=blk, sp=sp, par=par, wcnt=wcnt):
                        @pl.when(blk >= 2)
                        def _dr():
                            drain_scatter(sp)

                        ev = aux[pl.ds(blk * LANES, LANES)]
                        offv = (lax.shift_right_logical(ev, 14)
                                & (WELEM - 1))
                        bv = ev & 16383
                        pos = jnp.broadcast_to(blk * LANES, (LANES,)) + lane
                        sidx[sp, pl.ds(0, LANES)] = jnp.where(
                            pos < wcnt, bv, dummy_row)
                        for l in range(LANES):
                            go = jnp.broadcast_to(offv[l], (LANES,))
                            c0 = plsc.load_gather(slab.at[par], [f_lo, go])
                            c1 = plsc.load_gather(slab.at[par], [f_hi, go])
                            stage[sp, l, pl.ds(0, LANES)] = c0
                            stage[sp, l, pl.ds(LANES, LANES)] = c1
                        pltpu.async_copy(stage.at[sp],
                                         dst_hbm.at[sidx.at[sp]], sems_s[sp])
                return 0

            lax.fori_loop(0, (nblk + 1) >> 1, blk_pair, 0)

            @pl.when(w + 2 < NWIN)
            def _refill(w=w, par=par):
                fire(w + 2, par)
            prev = nblk
        return prev

    prev = lax.fori_loop(0, NWIN // 2, win_pair, jnp.int32(0))

    @pl.when(prev >= 1)
    def _fd0():
        drain_scatter(0)

    @pl.when(prev >= 2)
    def _fd1():
        drain_scatter(1)


def _k1_body(user_hbm, item_hbm, uT_hbm, iT_hbm, pr_hbm, qr_hbm,
             aux, uo, slab, stage, sidx,
             sem_f, sem_f2, sem_s, sem_s2):
    wid = lax.axis_index("s") * NC + lax.axis_index("c")
    w0 = wid * RNG
    _sweep(user_hbm, uT_hbm, pr_hbm, aux, uo, slab, stage, sidx,
           sem_f, sem_f2, sem_s, sem_s2, wid, w0)
    _sweep(item_hbm, iT_hbm, qr_hbm, aux, uo, slab, stage, sidx,
           sem_f, sem_f2, sem_s, sem_s2, wid, w0)


def _k2_body(pr_hbm, qr_hbm, hw_hbm, hb_hbm, out_hbm,
             ch_u, ch_i, w_v, b_v, out_v, sem0, sem1):
    wid = lax.axis_index("s") * NC + lax.axis_index("c")
    base = wid * B_PER_W
    sems = (sem0, sem1)
    pltpu.sync_copy(hw_hbm.at[0], w_v)
    pltpu.sync_copy(hb_hbm, b_v.at[pl.ds(0, 1)])
    w0 = w_v[pl.ds(0, LANES)]
    w1 = w_v[pl.ds(LANES, LANES)]
    b = b_v[pl.ds(0, LANES)][0]
    lane = jnp.arange(LANES, dtype=jnp.int32)
    NCH = B_PER_W // 128

    def fire(c):
        p = c & 1
        return (
            pltpu.async_copy(pr_hbm.at[pl.ds(base + c * 128, 128)],
                             ch_u.at[p], sems[p]),
            pltpu.async_copy(qr_hbm.at[pl.ds(base + c * 128, 128)],
                             ch_i.at[p], sems[p]),
        )

    inflight = {0: fire(0)}
    for c in range(NCH):
        if c + 1 < NCH:
            inflight[c + 1] = fire(c + 1)
        for cp in inflight.pop(c):
            cp.wait()
        p = c & 1

        def group(g, _, c=c, p=p):
            acc = jnp.zeros((LANES,), jnp.float32)
            for j in range(LANES):
                r = g * LANES + j
                p0 = ch_u[p, r, pl.ds(0, LANES)]
                p1 = ch_u[p, r, pl.ds(LANES, LANES)]
                q0 = ch_i[p, r, pl.ds(0, LANES)]
                q1 = ch_i[p, r, pl.ds(LANES, LANES)]
                s = p0 * q0 * w0 + p1 * q1 * w1
                acc = jnp.where(lane == j, jnp.sum(s), acc)
            out_v[pl.ds(c * 128 + g * LANES, LANES)] = acc + b
            return 0

        lax.fori_loop(0, 128 // LANES, group, 0)

    pltpu.sync_copy(out_v, out_hbm.at[pl.ds(base, B_PER_W)])


@jax.jit
def _gmf(user, item, user_emb, item_emb, h_w, h_b):
    mesh = plsc.VectorSubcoreMesh(core_axis_name="c", subcore_axis_name="s")
    k1 = functools.partial(
        pl.kernel,
        mesh=mesh,
        out_type=(
            jax.ShapeDtypeStruct((STAGE_ROWS, TCOL), jnp.float32),
            jax.ShapeDtypeStruct((STAGE_ROWS, TCOL), jnp.float32),
        ),
        scratch_types=[
            pltpu.VMEM((CAP,), jnp.int32),
            pltpu.VMEM((CAP,), jnp.int32),
            pltpu.VMEM((2, N_FACTORS, PITCH), jnp.float32),
            pltpu.VMEM((2, LANES, TCOL), jnp.float32),
            pltpu.VMEM((2, LANES), jnp.int32),
            pltpu.SemaphoreType.DMA,
            pltpu.SemaphoreType.DMA,
            pltpu.SemaphoreType.DMA,
            pltpu.SemaphoreType.DMA,
        ],
        compiler_params=pltpu.CompilerParams(needs_layout_passes=False),
    )(_k1_body)
    pr, qr = k1(user, item, user_emb.T, item_emb.T)

    k2 = functools.partial(
        pl.kernel,
        mesh=mesh,
        out_type=jax.ShapeDtypeStruct((BATCH,), jnp.float32),
        scratch_types=[
            pltpu.VMEM((2, 128, TCOL), jnp.float32),
            pltpu.VMEM((2, 128, TCOL), jnp.float32),
            pltpu.VMEM((N_FACTORS,), jnp.float32),
            pltpu.VMEM((LANES,), jnp.float32),
            pltpu.VMEM((B_PER_W,), jnp.float32),
            pltpu.SemaphoreType.DMA,
            pltpu.SemaphoreType.DMA,
        ],
        compiler_params=pltpu.CompilerParams(needs_layout_passes=False),
    )(_k2_body)
    return k2(pr, qr, h_w, h_b)


def kernel(user, item, user_emb, item_emb, h_w, h_b):
    return _gmf(user, item, user_emb, item_emb, h_w, h_b)
